# Initial kernel scaffold; baseline (speedup 1.0000x reference)
import jax
import jax.numpy as jnp
from jax.experimental import pallas as pl


def kernel(x, pre_w0, pre_w1, pre_w2, W_proj, a_l, a_r, Wm, gate_w, gate_b, bn_gamma, bn_beta, edge_index0, edge_index1, edge_index2):
    raise NotImplementedError("write your pallas kernel here")


# speedup vs baseline: 28.7818x; 28.7818x over previous
import functools

import jax
import jax.numpy as jnp
from jax import lax
from jax.experimental import pallas as pl
from jax.experimental.pallas import tpu as pltpu
from jax.experimental.pallas import tpu_sc as plsc

N = 10000
E = 320000
FEAT = 128
HID = 32
HEADS = 4
ND = 3
C_OUT = HID * (ND + 1)
TLSE = 18.0

NTILES = 32
EPT = E // NTILES
CH = 80
NCH = EPT // CH
ZR = 128
NZCH = N // ZR
ZREM = N - NZCH * ZR
ZPT = (NZCH + 15) // 16


def _pre_body(x_ref, wcat_ref, w2cat_ref, y_ref, q_ref, ea_ref, m_ref):
    x = x_ref[...]
    y = jnp.dot(x, wcat_ref[...], preferred_element_type=jnp.float32)
    y_ref[...] = y
    proj = y[:, 0:HID]
    ea_ref[...] = jnp.dot(proj, w2cat_ref[...], preferred_element_type=jnp.float32)
    for g in range(ND):
        xm = y[:, HID + HID * g:2 * HID + HID * g]
        mcol = jnp.max(xm, axis=0)
        m_ref[g] = mcol
        q_ref[:, HID * g:HID * (g + 1)] = jnp.exp(TLSE * (xm - mcol[None, :]))


_pre_call = pl.pallas_call(
    _pre_body,
    out_shape=[
        jax.ShapeDtypeStruct((N, 152), jnp.float32),
        jax.ShapeDtypeStruct((N, ND * HID), jnp.float32),
        jax.ShapeDtypeStruct((N, ND * 8), jnp.float32),
        jax.ShapeDtypeStruct((ND, HID), jnp.float32),
    ],
)


def _s1_body(src_hbm, dst_hbm, pw_hbm, srcrow_hbm, ertab_hbm,
             aout_hbm, bout_hbm, exout_hbm,
             srcv, dstv, pwv, srow, erow, bbuf, zbuf48, zbuf16, acc_a, acc_b):
    cid = lax.axis_index("c")
    sid = lax.axis_index("s")
    w = sid * 2 + cid
    zv = jnp.zeros((16,), jnp.float32)

    def zinit(r, _):
        zbuf48[r, 0:16] = zv
        zbuf48[r, 16:32] = zv
        zbuf48[r, 32:48] = zv
        zbuf16[r, 0:16] = zv
        return 0
    lax.fori_loop(0, ZR, zinit, 0)

    onehot4 = jnp.where(lax.iota(jnp.int32, 16) == 4, 1.0, 0.0).astype(jnp.float32)
    def binit(r, _):
        bbuf[r, 0:16] = onehot4
        return 0
    lax.fori_loop(0, CH, binit, 0)

    for g in range(ND):
        def zcp(t, _):
            cidx = sid * ZPT + t
            @pl.when(cidx < NZCH)
            def _():
                pltpu.sync_copy(zbuf48, acc_a.at[pl.ds(cidx * ZR, ZR)])
                pltpu.sync_copy(zbuf16, acc_b.at[pl.ds(cidx * ZR, ZR)])
            return 0
        lax.fori_loop(0, ZPT, zcp, 0)

        @pl.when(sid == 0)
        def _():
            pltpu.sync_copy(zbuf48.at[pl.ds(0, ZREM)],
                            acc_a.at[pl.ds(NZCH * ZR, ZREM)])
            pltpu.sync_copy(zbuf16.at[pl.ds(0, ZREM)],
                            acc_b.at[pl.ds(NZCH * ZR, ZREM)])

        plsc.subcore_barrier()

        ebase = g * E + w * EPT

        def chunk(j, _):
            base = pl.multiple_of(ebase + j * CH, 8)
            pltpu.sync_copy(src_hbm.at[pl.ds(base, CH)], srcv)
            pltpu.sync_copy(dst_hbm.at[pl.ds(base, CH)], dstv)
            pltpu.sync_copy(pw_hbm.at[pl.ds(base, CH)], pwv)
            pltpu.sync_copy(srcrow_hbm.at[srcv], srow)
            pltpu.sync_copy(ertab_hbm.at[dstv], erow)
            for jj in range(CH // 16):
                rows = lax.iota(jnp.int32, 16) + (jj * 16)
                pwvec = pwv[pl.ds(jj * 16, 16)]
                for h in range(HEADS):
                    ch = jnp.full((16,), h, jnp.int32)
                    elh = plsc.load_gather(srow, [rows, ch + 36])
                    erh = plsc.load_gather(erow, [rows, ch])
                    e = elh + erh
                    e = jnp.maximum(e, 0.2 * e) * pwvec
                    plsc.store_scatter(bbuf, [rows, ch], jnp.exp(e))
            if g > 0:
                for jj in range(CH // 16):
                    sl = pl.ds(jj * 16, 16)
                    dstv[sl] = dstv[sl] - (g * N)
            pltpu.sync_copy(srow, acc_a.at[dstv], add=True)
            pltpu.sync_copy(bbuf, acc_b.at[dstv], add=True)
            pltpu.sync_copy(bbuf, exout_hbm.at[pl.ds(base, CH)])
            return 0

        lax.fori_loop(0, NCH, chunk, 0)

        plsc.subcore_barrier()

        @pl.when(sid == 0)
        def _():
            pltpu.sync_copy(acc_a, aout_hbm.at[cid, g])
            pltpu.sync_copy(acc_b, bout_hbm.at[cid, g])

        plsc.subcore_barrier()


_s1_call = pl.kernel(
    _s1_body,
    out_type=[
        jax.ShapeDtypeStruct((2, ND, N, 48), jnp.float32),
        jax.ShapeDtypeStruct((2, ND, N, 16), jnp.float32),
        jax.ShapeDtypeStruct((ND * E, 16), jnp.float32),
    ],
    mesh=plsc.VectorSubcoreMesh(core_axis_name="c", subcore_axis_name="s"),
    compiler_params=pltpu.CompilerParams(needs_layout_passes=False, use_tc_tiling_on_sc=False),
    scratch_types=[
        pltpu.VMEM((CH,), jnp.int32),
        pltpu.VMEM((CH,), jnp.int32),
        pltpu.VMEM((CH,), jnp.float32),
        pltpu.VMEM((CH, 48), jnp.float32),
        pltpu.VMEM((CH, 16), jnp.float32),
        pltpu.VMEM((CH, 16), jnp.float32),
        pltpu.VMEM((ZR, 48), jnp.float32),
        pltpu.VMEM((ZR, 16), jnp.float32),
        pltpu.VMEM_SHARED((N, 48), jnp.float32),
        pltpu.VMEM_SHARED((N, 16), jnp.float32),
    ],
)


_GB = 1000


def _gate_body(apart_ref, bpart_ref, m_ref, xg1_ref, gw2_ref, gb_ref, c_ref):
    for i in range(ND):
        a = apart_ref[0, i] + apart_ref[1, i]
        b = bpart_ref[0, i] + bpart_ref[1, i]
        sq = a[:, 0:HID]
        xgs = a[:, HID:HID + HEADS]
        exsum = b[:, 0:HEADS]
        deg = b[:, 4:5]
        mx = m_ref[i][None, :] + jnp.log(jnp.maximum(sq, 1e-43)) * (1.0 / TLSE)
        mx = jnp.where(deg > 0.0, mx, 0.0)
        mean4 = xgs / jnp.maximum(deg, 1.0)
        logits = (xg1_ref[i]
                  + jnp.dot(mx, gw2_ref[i], preferred_element_type=jnp.float32)
                  + mean4 + gb_ref[i][None, :])
        g = jax.nn.sigmoid(logits)
        c_ref[i] = g / (4.0 * (exsum + 1e-9))


_gate_call = pl.pallas_call(
    _gate_body,
    grid=(N // _GB,),
    in_specs=[
        pl.BlockSpec((2, ND, _GB, 48), lambda i: (0, 0, i, 0)),
        pl.BlockSpec((2, ND, _GB, 16), lambda i: (0, 0, i, 0)),
        pl.BlockSpec((ND, HID), lambda i: (0, 0)),
        pl.BlockSpec((ND, _GB, HEADS), lambda i: (0, i, 0)),
        pl.BlockSpec((ND, HID, HEADS), lambda i: (0, 0, 0)),
        pl.BlockSpec((ND, HEADS), lambda i: (0, 0)),
    ],
    out_specs=[pl.BlockSpec((ND, _GB, HEADS), lambda i: (0, i, 0))],
    out_shape=[jax.ShapeDtypeStruct((ND, N, HEADS), jnp.float32)],
)


def _s2_body(src_hbm, dst_hbm, ztab_hbm, ctab_hbm, exbuf_hbm,
             accout_hbm,
             srcv, dstv, zrow, crow, exrow, zbuf32, acc):
    cid = lax.axis_index("c")
    sid = lax.axis_index("s")
    w = sid * 2 + cid
    zv = jnp.zeros((16,), jnp.float32)

    def zinit(r, _):
        zbuf32[r, 0:16] = zv
        zbuf32[r, 16:32] = zv
        return 0
    lax.fori_loop(0, ZR, zinit, 0)

    for g in range(ND):
        def zcp(t, _):
            cidx = sid * ZPT + t
            @pl.when(cidx < NZCH)
            def _():
                pltpu.sync_copy(zbuf32, acc.at[pl.ds(cidx * ZR, ZR)])
            return 0
        lax.fori_loop(0, ZPT, zcp, 0)

        @pl.when(sid == 0)
        def _():
            pltpu.sync_copy(zbuf32.at[pl.ds(0, ZREM)],
                            acc.at[pl.ds(NZCH * ZR, ZREM)])

        plsc.subcore_barrier()

        ebase = g * E + w * EPT

        def chunk(j, _):
            base = pl.multiple_of(ebase + j * CH, 8)
            pltpu.sync_copy(src_hbm.at[pl.ds(base, CH)], srcv)
            pltpu.sync_copy(dst_hbm.at[pl.ds(base, CH)], dstv)
            pltpu.sync_copy(exbuf_hbm.at[pl.ds(base, CH)], exrow)
            pltpu.sync_copy(ztab_hbm.at[srcv], zrow)
            pltpu.sync_copy(ctab_hbm.at[dstv], crow)
            for jj in range(CH // 16):
                rows = lax.iota(jnp.int32, 16) + (jj * 16)
                csum = jnp.zeros((16,), jnp.float32)
                for h in range(HEADS):
                    ch = jnp.full((16,), h, jnp.int32)
                    csum = csum + (plsc.load_gather(exrow, [rows, ch])
                                   * plsc.load_gather(crow, [rows, ch]))
                for col in range(HID):
                    cc = jnp.full((16,), col, jnp.int32)
                    v = plsc.load_gather(zrow, [rows, cc])
                    plsc.store_scatter(zrow, [rows, cc], v * csum)
            if g > 0:
                for jj in range(CH // 16):
                    sl = pl.ds(jj * 16, 16)
                    dstv[sl] = dstv[sl] - (g * N)
            pltpu.sync_copy(zrow, acc.at[dstv], add=True)
            return 0

        lax.fori_loop(0, NCH, chunk, 0)

        plsc.subcore_barrier()

        @pl.when(sid == 0)
        def _():
            pltpu.sync_copy(acc, accout_hbm.at[cid, g])

        plsc.subcore_barrier()


_s2_call = pl.kernel(
    _s2_body,
    out_type=[jax.ShapeDtypeStruct((2, ND, N, HID), jnp.float32)],
    mesh=plsc.VectorSubcoreMesh(core_axis_name="c", subcore_axis_name="s"),
    compiler_params=pltpu.CompilerParams(needs_layout_passes=False, use_tc_tiling_on_sc=False),
    scratch_types=[
        pltpu.VMEM((CH,), jnp.int32),
        pltpu.VMEM((CH,), jnp.int32),
        pltpu.VMEM((CH, HID), jnp.float32),
        pltpu.VMEM((CH, 16), jnp.float32),
        pltpu.VMEM((CH, 16), jnp.float32),
        pltpu.VMEM((ZR, HID), jnp.float32),
        pltpu.VMEM_SHARED((N, HID), jnp.float32),
    ],
)


def _stats_body(proj_ref, accp_ref, h_ref, stats_ref):
    i = pl.program_id(0)
    parts = [proj_ref[...]]
    for g in range(ND):
        parts.append(accp_ref[0, g] + accp_ref[1, g])
    h = jnp.concatenate(parts, axis=1)
    h_ref[0] = h

    @pl.when(i == 0)
    def _():
        stats_ref[...] = jnp.zeros((2, C_OUT), jnp.float32)
    stats_ref[0:1] = stats_ref[0:1] + jnp.sum(h, axis=0, keepdims=True)
    stats_ref[1:2] = stats_ref[1:2] + jnp.sum(h * h, axis=0, keepdims=True)


_stats_call = pl.pallas_call(
    _stats_body,
    grid=(N // _GB,),
    in_specs=[
        pl.BlockSpec((_GB, HID), lambda i: (i, 0)),
        pl.BlockSpec((2, ND, _GB, HID), lambda i: (0, 0, i, 0)),
    ],
    out_specs=[
        pl.BlockSpec((1, _GB, C_OUT), lambda i: (0, i, 0)),
        pl.BlockSpec((2, C_OUT), lambda i: (0, 0)),
    ],
    out_shape=[
        jax.ShapeDtypeStruct((1, N, C_OUT), jnp.float32),
        jax.ShapeDtypeStruct((2, C_OUT), jnp.float32),
    ],
)


def _bn_body(h_ref, stats_ref, gamma_ref, beta_ref, out_ref):
    mean = stats_ref[0:1] * (1.0 / N)
    var = stats_ref[1:2] * (1.0 / N) - mean * mean
    hn = (h_ref[0] - mean) * lax.rsqrt(var + 1e-5)
    out_ref[0] = hn * gamma_ref[...][None, :] + beta_ref[...][None, :]


_bn_call = pl.pallas_call(
    _bn_body,
    grid=(N // _GB,),
    in_specs=[
        pl.BlockSpec((1, _GB, C_OUT), lambda i: (0, i, 0)),
        pl.BlockSpec((2, C_OUT), lambda i: (0, 0)),
        pl.BlockSpec((C_OUT,), lambda i: (0,)),
        pl.BlockSpec((C_OUT,), lambda i: (0,)),
    ],
    out_specs=[pl.BlockSpec((1, _GB, C_OUT), lambda i: (0, i, 0))],
    out_shape=[jax.ShapeDtypeStruct((1, N, C_OUT), jnp.float32)],
)


def kernel(x, pre_w0, pre_w1, pre_w2, W_proj, a_l, a_r, Wm, gate_w, gate_b,
           bn_gamma, bn_beta, edge_index0, edge_index1, edge_index2):
    alt = jnp.transpose(a_l, (0, 2, 1))
    art = jnp.transpose(a_r, (0, 2, 1))
    gw1 = gate_w[:, 0:FEAT, :]
    gw2 = gate_w[:, FEAT:FEAT + HID, :]
    gw3 = gate_w[:, FEAT + HID:, :]

    wcat = jnp.concatenate(
        [W_proj, Wm[0], Wm[1], Wm[2],
         gw3[0], gw1[0], gw3[1], gw1[1], gw3[2], gw1[2]], axis=1)
    w2cat = jnp.concatenate(
        [alt[0], art[0], alt[1], art[1], alt[2], art[2]], axis=1)

    y, qcat, ea, m = _pre_call(x, wcat, w2cat)
    proj = y[:, 0:HID]

    zpad8 = jnp.zeros((N, 8), jnp.float32)
    srcrow = jnp.concatenate([
        jnp.concatenate([qcat[:, HID * g:HID * (g + 1)],
                         y[:, FEAT + 8 * g:FEAT + 8 * g + 4],
                         ea[:, 8 * g:8 * g + 4], zpad8], axis=1)
        for g in range(ND)], axis=0)
    zpad12 = jnp.zeros((N, 12), jnp.float32)
    ertab = jnp.concatenate([
        jnp.concatenate([ea[:, 8 * g + 4:8 * g + 8], zpad12], axis=1)
        for g in range(ND)], axis=0)
    xg1 = jnp.stack([y[:, FEAT + 8 * g + 4:FEAT + 8 * g + 8] for g in range(ND)])

    offs = jnp.arange(ND, dtype=jnp.int32)[:, None] * N
    src_all = (jnp.stack([edge_index0[0], edge_index1[0], edge_index2[0]]) + offs).reshape(-1)
    dst_all = (jnp.stack([edge_index0[1], edge_index1[1], edge_index2[1]]) + offs).reshape(-1)
    pw_all = jnp.concatenate([pre_w0, pre_w1, pre_w2])

    apart, bpart, exbuf = _s1_call(src_all, dst_all, pw_all, srcrow, ertab)

    (coeff,) = _gate_call(apart, bpart, m, xg1, gw2, gate_b)
    ctab = jnp.concatenate(
        [coeff.reshape(ND * N, HEADS), jnp.zeros((ND * N, 12), jnp.float32)], axis=1)
    ztab = jnp.tile(proj, (ND, 1))

    (accp,) = _s2_call(src_all, dst_all, ztab, ctab, exbuf)

    h, stats = _stats_call(proj, accp)
    (out,) = _bn_call(h, stats, bn_gamma, bn_beta)
    return out


# speedup vs baseline: 40.3679x; 1.4025x over previous
import functools

import jax
import jax.numpy as jnp
from jax import lax
from jax.experimental import pallas as pl
from jax.experimental.pallas import tpu as pltpu
from jax.experimental.pallas import tpu_sc as plsc

N = 10000
E = 320000
FEAT = 128
HID = 32
HEADS = 4
ND = 3
C_OUT = HID * (ND + 1)
TLSE = 18.0

NTILES = 32
EPT = E // NTILES
CH = 80
NCH = EPT // CH
ZR = 128
NZCH = N // ZR
ZREM = N - NZCH * ZR
ZPT = (NZCH + 15) // 16


def _pre_body(x_ref, wcat_ref, w2cat_ref, y_ref, q_ref, ea_ref, m_ref):
    x = x_ref[...]
    y = jnp.dot(x, wcat_ref[...], preferred_element_type=jnp.float32)
    y_ref[...] = y
    proj = y[:, 0:HID]
    ea_ref[...] = jnp.dot(proj, w2cat_ref[...], preferred_element_type=jnp.float32)
    for g in range(ND):
        xm = y[:, HID + HID * g:2 * HID + HID * g]
        mcol = jnp.max(xm, axis=0)
        m_ref[g] = mcol
        q_ref[:, HID * g:HID * (g + 1)] = jnp.exp(TLSE * (xm - mcol[None, :]))


_pre_call = pl.pallas_call(
    _pre_body,
    out_shape=[
        jax.ShapeDtypeStruct((N, 152), jnp.float32),
        jax.ShapeDtypeStruct((N, ND * HID), jnp.float32),
        jax.ShapeDtypeStruct((N, ND * 8), jnp.float32),
        jax.ShapeDtypeStruct((ND, HID), jnp.float32),
    ],
)


NB = 5


def _s1_body(*refs):
    (src_hbm, dst_hbm, pw_hbm, srcrow_hbm, ertab_hbm,
     aout_hbm, bout_hbm, exout_hbm) = refs[:8]
    r = list(refs[8:])
    srcv = r[0:NB]
    dstv = r[NB:2 * NB]
    pwv = r[2 * NB:3 * NB]
    srow = r[3 * NB:4 * NB]
    erow = r[4 * NB:5 * NB]
    bbuf = r[5 * NB:6 * NB]
    exb4 = r[6 * NB:7 * NB]
    zbuf48, zbuf16, acc_a, acc_b = r[7 * NB:7 * NB + 4]
    semi = r[7 * NB + 4:7 * NB + 4 + NB]
    semg = r[7 * NB + 4 + NB:7 * NB + 4 + 2 * NB]
    sems = r[7 * NB + 4 + 2 * NB:7 * NB + 4 + 3 * NB]

    cid = lax.axis_index("c")
    sid = lax.axis_index("s")
    w = sid * 2 + cid
    zv = jnp.zeros((16,), jnp.float32)

    def zinit(rr, _):
        zbuf48[rr, 0:16] = zv
        zbuf48[rr, 16:32] = zv
        zbuf48[rr, 32:48] = zv
        zbuf16[rr, 0:16] = zv
        return 0
    lax.fori_loop(0, ZR, zinit, 0)

    onehot4 = jnp.where(lax.iota(jnp.int32, 16) == 4, 1.0, 0.0).astype(jnp.float32)
    for b in range(NB):
        def binit(rr, _, _b=b):
            bbuf[_b][rr, 0:16] = onehot4
            return 0
        lax.fori_loop(0, CH, binit, 0)

    for g in range(ND):
        def zcp(t, _):
            cidx = sid * ZPT + t
            @pl.when(cidx < NZCH)
            def _():
                pltpu.sync_copy(zbuf48, acc_a.at[pl.ds(cidx * ZR, ZR)])
                pltpu.sync_copy(zbuf16, acc_b.at[pl.ds(cidx * ZR, ZR)])
            return 0
        lax.fori_loop(0, ZPT, zcp, 0)

        @pl.when(sid == 0)
        def _():
            pltpu.sync_copy(zbuf48.at[pl.ds(0, ZREM)],
                            acc_a.at[pl.ds(NZCH * ZR, ZREM)])
            pltpu.sync_copy(zbuf16.at[pl.ds(0, ZREM)],
                            acc_b.at[pl.ds(NZCH * ZR, ZREM)])

        plsc.subcore_barrier()

        ebase = g * E + w * EPT

        def jloop(j, _):
            b = 0
            base = pl.multiple_of(ebase + j * CH, 8)
            di = (
                pltpu.async_copy(src_hbm.at[pl.ds(base, CH)], srcv[b], semi[0]),
                pltpu.async_copy(dst_hbm.at[pl.ds(base, CH)], dstv[b], semi[1]),
                pltpu.async_copy(pw_hbm.at[pl.ds(base, CH)], pwv[b], semi[2]),
            )
            for d in di:
                d.wait()
            dg = (
                pltpu.async_copy(srcrow_hbm.at[srcv[b]], srow[b], semg[0]),
                pltpu.async_copy(ertab_hbm.at[dstv[b]], erow[b], semg[1]),
            )
            for d in dg:
                d.wait()
            for jj in range(CH // 16):
                rows = lax.iota(jnp.int32, 16) + (jj * 16)
                pwvec = pwv[b][pl.ds(jj * 16, 16)]
                for h in range(HEADS):
                    ch = jnp.full((16,), h, jnp.int32)
                    elh = plsc.load_gather(srow[b], [rows, ch + 36])
                    erh = plsc.load_gather(erow[b], [rows, ch])
                    e = elh + erh
                    e = jnp.maximum(e, 0.2 * e) * pwvec
                    exv = jnp.exp(e)
                    plsc.store_scatter(bbuf[b], [rows, ch], exv)
                    plsc.store_scatter(exb4[b], [rows, ch], exv)
            for jj in range(CH // 16):
                sl = pl.ds(jj * 16, 16)
                dstv[b][sl] = dstv[b][sl] - (g * N)
            ds = (
                pltpu.async_copy(srow[b], acc_a.at[dstv[b]], sems[0], add=True),
                pltpu.async_copy(bbuf[b], acc_b.at[dstv[b]], sems[1], add=True),
                pltpu.async_copy(exb4[b], exout_hbm.at[pl.ds(base, CH)], sems[2]),
            )
            for d in ds:
                d.wait()
            return 0

        lax.fori_loop(0, NCH, jloop, 0)

        plsc.subcore_barrier()

        @pl.when(sid == 0)
        def _():
            pltpu.sync_copy(acc_a, aout_hbm.at[cid, g])
            pltpu.sync_copy(acc_b, bout_hbm.at[cid, g])

        plsc.subcore_barrier()


_s1_call = pl.kernel(
    _s1_body,
    out_type=[
        jax.ShapeDtypeStruct((2, ND, N, 48), jnp.float32),
        jax.ShapeDtypeStruct((2, ND, N, 16), jnp.float32),
        jax.ShapeDtypeStruct((ND * E, HEADS), jnp.float32),
    ],
    mesh=plsc.VectorSubcoreMesh(core_axis_name="c", subcore_axis_name="s"),
    compiler_params=pltpu.CompilerParams(needs_layout_passes=False, use_tc_tiling_on_sc=False),
    scratch_types=(
        [pltpu.VMEM((CH,), jnp.int32)] * NB
        + [pltpu.VMEM((CH,), jnp.int32)] * NB
        + [pltpu.VMEM((CH,), jnp.float32)] * NB
        + [pltpu.VMEM((CH, 48), jnp.float32)] * NB
        + [pltpu.VMEM((CH, 16), jnp.float32)] * NB
        + [pltpu.VMEM((CH, 16), jnp.float32)] * NB
        + [pltpu.VMEM((CH, HEADS), jnp.float32)] * NB
        + [pltpu.VMEM((ZR, 48), jnp.float32),
           pltpu.VMEM((ZR, 16), jnp.float32),
           pltpu.VMEM_SHARED((N, 48), jnp.float32),
           pltpu.VMEM_SHARED((N, 16), jnp.float32)]
        + [pltpu.SemaphoreType.DMA] * (3 * NB)
    ),
)


_GB = 1000


def _gate_body(apart_ref, bpart_ref, m_ref, xg1_ref, gw2_ref, gb_ref, c_ref):
    for i in range(ND):
        a = apart_ref[0, i] + apart_ref[1, i]
        b = bpart_ref[0, i] + bpart_ref[1, i]
        sq = a[:, 0:HID]
        xgs = a[:, HID:HID + HEADS]
        exsum = b[:, 0:HEADS]
        deg = b[:, 4:5]
        mx = m_ref[i][None, :] + jnp.log(jnp.maximum(sq, 1e-43)) * (1.0 / TLSE)
        mx = jnp.where(deg > 0.0, mx, 0.0)
        mean4 = xgs / jnp.maximum(deg, 1.0)
        logits = (xg1_ref[i]
                  + jnp.dot(mx, gw2_ref[i], preferred_element_type=jnp.float32)
                  + mean4 + gb_ref[i][None, :])
        g = jax.nn.sigmoid(logits)
        c_ref[i] = g / (4.0 * (exsum + 1e-9))


_gate_call = pl.pallas_call(
    _gate_body,
    grid=(N // _GB,),
    in_specs=[
        pl.BlockSpec((2, ND, _GB, 48), lambda i: (0, 0, i, 0)),
        pl.BlockSpec((2, ND, _GB, 16), lambda i: (0, 0, i, 0)),
        pl.BlockSpec((ND, HID), lambda i: (0, 0)),
        pl.BlockSpec((ND, _GB, HEADS), lambda i: (0, i, 0)),
        pl.BlockSpec((ND, HID, HEADS), lambda i: (0, 0, 0)),
        pl.BlockSpec((ND, HEADS), lambda i: (0, 0)),
    ],
    out_specs=[pl.BlockSpec((ND, _GB, HEADS), lambda i: (0, i, 0))],
    out_shape=[jax.ShapeDtypeStruct((ND, N, HEADS), jnp.float32)],
)


def _s2_body(*refs):
    (src_hbm, dst_hbm, ztab_hbm, ctab_hbm, exbuf_hbm, accout_hbm) = refs[:6]
    r = list(refs[6:])
    srcv = r[0:NB]
    dstv = r[NB:2 * NB]
    zrow = r[2 * NB:3 * NB]
    crow = r[3 * NB:4 * NB]
    exrow = r[4 * NB:5 * NB]
    zbuf32, acc = r[5 * NB:5 * NB + 2]
    semi = r[5 * NB + 2:5 * NB + 2 + NB]
    semg = r[5 * NB + 2 + NB:5 * NB + 2 + 2 * NB]
    sems = r[5 * NB + 2 + 2 * NB:5 * NB + 2 + 3 * NB]

    cid = lax.axis_index("c")
    sid = lax.axis_index("s")
    w = sid * 2 + cid
    zv = jnp.zeros((16,), jnp.float32)

    def zinit(rr, _):
        zbuf32[rr, 0:16] = zv
        zbuf32[rr, 16:32] = zv
        return 0
    lax.fori_loop(0, ZR, zinit, 0)

    for g in range(ND):
        def zcp(t, _):
            cidx = sid * ZPT + t
            @pl.when(cidx < NZCH)
            def _():
                pltpu.sync_copy(zbuf32, acc.at[pl.ds(cidx * ZR, ZR)])
            return 0
        lax.fori_loop(0, ZPT, zcp, 0)

        @pl.when(sid == 0)
        def _():
            pltpu.sync_copy(zbuf32.at[pl.ds(0, ZREM)],
                            acc.at[pl.ds(NZCH * ZR, ZREM)])

        plsc.subcore_barrier()

        ebase = g * E + w * EPT

        def jloop(j, _):
            b = 0
            base = pl.multiple_of(ebase + j * CH, 8)
            di = (
                pltpu.async_copy(src_hbm.at[pl.ds(base, CH)], srcv[b], semi[0]),
                pltpu.async_copy(dst_hbm.at[pl.ds(base, CH)], dstv[b], semi[1]),
                pltpu.async_copy(exbuf_hbm.at[pl.ds(base, CH)], exrow[b], semi[2]),
            )
            for d in di:
                d.wait()
            dg = (
                pltpu.async_copy(ztab_hbm.at[srcv[b]], zrow[b], semg[0]),
                pltpu.async_copy(ctab_hbm.at[dstv[b]], crow[b], semg[1]),
            )
            for d in dg:
                d.wait()
            for jj in range(CH // 16):
                rows = lax.iota(jnp.int32, 16) + (jj * 16)
                csum = jnp.zeros((16,), jnp.float32)
                for h in range(HEADS):
                    ch = jnp.full((16,), h, jnp.int32)
                    csum = csum + (plsc.load_gather(exrow[b], [rows, ch])
                                   * plsc.load_gather(crow[b], [rows, ch]))
                for col in range(HID):
                    cc = jnp.full((16,), col, jnp.int32)
                    v = plsc.load_gather(zrow[b], [rows, cc])
                    plsc.store_scatter(zrow[b], [rows, cc], v * csum)
            for jj in range(CH // 16):
                sl = pl.ds(jj * 16, 16)
                dstv[b][sl] = dstv[b][sl] - (g * N)
            pltpu.async_copy(zrow[b], acc.at[dstv[b]], sems[0], add=True).wait()
            return 0

        lax.fori_loop(0, NCH, jloop, 0)

        plsc.subcore_barrier()

        @pl.when(sid == 0)
        def _():
            pltpu.sync_copy(acc, accout_hbm.at[cid, g])

        plsc.subcore_barrier()


_s2_call = pl.kernel(
    _s2_body,
    out_type=[jax.ShapeDtypeStruct((2, ND, N, HID), jnp.float32)],
    mesh=plsc.VectorSubcoreMesh(core_axis_name="c", subcore_axis_name="s"),
    compiler_params=pltpu.CompilerParams(needs_layout_passes=False, use_tc_tiling_on_sc=False),
    scratch_types=(
        [pltpu.VMEM((CH,), jnp.int32)] * NB
        + [pltpu.VMEM((CH,), jnp.int32)] * NB
        + [pltpu.VMEM((CH, HID), jnp.float32)] * NB
        + [pltpu.VMEM((CH, 16), jnp.float32)] * NB
        + [pltpu.VMEM((CH, HEADS), jnp.float32)] * NB
        + [pltpu.VMEM((ZR, HID), jnp.float32),
           pltpu.VMEM_SHARED((N, HID), jnp.float32)]
        + [pltpu.SemaphoreType.DMA] * (3 * NB)
    ),
)


def _stats_body(proj_ref, accp_ref, h_ref, stats_ref):
    i = pl.program_id(0)
    parts = [proj_ref[...]]
    for g in range(ND):
        parts.append(accp_ref[0, g] + accp_ref[1, g])
    h = jnp.concatenate(parts, axis=1)
    h_ref[0] = h

    @pl.when(i == 0)
    def _():
        stats_ref[...] = jnp.zeros((2, C_OUT), jnp.float32)
    stats_ref[0:1] = stats_ref[0:1] + jnp.sum(h, axis=0, keepdims=True)
    stats_ref[1:2] = stats_ref[1:2] + jnp.sum(h * h, axis=0, keepdims=True)


_stats_call = pl.pallas_call(
    _stats_body,
    grid=(N // _GB,),
    in_specs=[
        pl.BlockSpec((_GB, HID), lambda i: (i, 0)),
        pl.BlockSpec((2, ND, _GB, HID), lambda i: (0, 0, i, 0)),
    ],
    out_specs=[
        pl.BlockSpec((1, _GB, C_OUT), lambda i: (0, i, 0)),
        pl.BlockSpec((2, C_OUT), lambda i: (0, 0)),
    ],
    out_shape=[
        jax.ShapeDtypeStruct((1, N, C_OUT), jnp.float32),
        jax.ShapeDtypeStruct((2, C_OUT), jnp.float32),
    ],
)


def _bn_body(h_ref, stats_ref, gamma_ref, beta_ref, out_ref):
    mean = stats_ref[0:1] * (1.0 / N)
    var = stats_ref[1:2] * (1.0 / N) - mean * mean
    hn = (h_ref[0] - mean) * lax.rsqrt(var + 1e-5)
    out_ref[0] = hn * gamma_ref[...][None, :] + beta_ref[...][None, :]


_bn_call = pl.pallas_call(
    _bn_body,
    grid=(N // _GB,),
    in_specs=[
        pl.BlockSpec((1, _GB, C_OUT), lambda i: (0, i, 0)),
        pl.BlockSpec((2, C_OUT), lambda i: (0, 0)),
        pl.BlockSpec((C_OUT,), lambda i: (0,)),
        pl.BlockSpec((C_OUT,), lambda i: (0,)),
    ],
    out_specs=[pl.BlockSpec((1, _GB, C_OUT), lambda i: (0, i, 0))],
    out_shape=[jax.ShapeDtypeStruct((1, N, C_OUT), jnp.float32)],
)


def kernel(x, pre_w0, pre_w1, pre_w2, W_proj, a_l, a_r, Wm, gate_w, gate_b,
           bn_gamma, bn_beta, edge_index0, edge_index1, edge_index2):
    alt = jnp.transpose(a_l, (0, 2, 1))
    art = jnp.transpose(a_r, (0, 2, 1))
    gw1 = gate_w[:, 0:FEAT, :]
    gw2 = gate_w[:, FEAT:FEAT + HID, :]
    gw3 = gate_w[:, FEAT + HID:, :]

    wcat = jnp.concatenate(
        [W_proj, Wm[0], Wm[1], Wm[2],
         gw3[0], gw1[0], gw3[1], gw1[1], gw3[2], gw1[2]], axis=1)
    w2cat = jnp.concatenate(
        [alt[0], art[0], alt[1], art[1], alt[2], art[2]], axis=1)

    y, qcat, ea, m = _pre_call(x, wcat, w2cat)
    proj = y[:, 0:HID]

    zpad8 = jnp.zeros((N, 8), jnp.float32)
    srcrow = jnp.concatenate([
        jnp.concatenate([qcat[:, HID * g:HID * (g + 1)],
                         y[:, FEAT + 8 * g:FEAT + 8 * g + 4],
                         ea[:, 8 * g:8 * g + 4], zpad8], axis=1)
        for g in range(ND)], axis=0)
    zpad12 = jnp.zeros((N, 12), jnp.float32)
    ertab = jnp.concatenate([
        jnp.concatenate([ea[:, 8 * g + 4:8 * g + 8], zpad12], axis=1)
        for g in range(ND)], axis=0)
    xg1 = jnp.stack([y[:, FEAT + 8 * g + 4:FEAT + 8 * g + 8] for g in range(ND)])

    offs = jnp.arange(ND, dtype=jnp.int32)[:, None] * N
    src_all = (jnp.stack([edge_index0[0], edge_index1[0], edge_index2[0]]) + offs).reshape(-1)
    dst_all = (jnp.stack([edge_index0[1], edge_index1[1], edge_index2[1]]) + offs).reshape(-1)
    pw_all = jnp.concatenate([pre_w0, pre_w1, pre_w2])

    apart, bpart, exbuf = _s1_call(src_all, dst_all, pw_all, srcrow, ertab)

    (coeff,) = _gate_call(apart, bpart, m, xg1, gw2, gate_b)
    ctab = jnp.concatenate(
        [coeff.reshape(ND * N, HEADS), jnp.zeros((ND * N, 12), jnp.float32)], axis=1)
    ztab = jnp.tile(proj, (ND, 1))

    (accp,) = _s2_call(src_all, dst_all, ztab, ctab, exbuf)

    h, stats = _stats_call(proj, accp)
    (out,) = _bn_call(h, stats, bn_gamma, bn_beta)
    return out


# speedup vs baseline: 40.4083x; 1.0010x over previous
import jax
import jax.numpy as jnp
from jax import lax
from jax.experimental import pallas as pl
from jax.experimental.pallas import tpu as pltpu
from jax.experimental.pallas import tpu_sc as plsc

N = 10000
E = 320000
FEAT = 128
HID = 32
HEADS = 4
ND = 3
C_OUT = HID * (ND + 1)
TLSE = 18.0

NTILES = 32
EPT = E // NTILES
CH = 80
NCH = EPT // CH
ZR = 128
NZCH = N // ZR
ZREM = N - NZCH * ZR
ZPT = (NZCH + 15) // 16


def _pre_body(x_ref, wcat_ref, w2cat_ref, y_ref, q_ref, ea_ref, m_ref):
    x = x_ref[...]
    y = jnp.dot(x, wcat_ref[...], preferred_element_type=jnp.float32)
    y_ref[...] = y
    proj = y[:, 0:HID]
    ea_ref[...] = jnp.dot(proj, w2cat_ref[...], preferred_element_type=jnp.float32)
    for g in range(ND):
        xm = y[:, HID + HID * g:2 * HID + HID * g]
        mcol = jnp.max(xm, axis=0)
        m_ref[g] = mcol
        q_ref[:, HID * g:HID * (g + 1)] = jnp.exp(TLSE * (xm - mcol[None, :]))


_pre_call = pl.pallas_call(
    _pre_body,
    out_shape=[
        jax.ShapeDtypeStruct((N, 152), jnp.float32),
        jax.ShapeDtypeStruct((N, ND * HID), jnp.float32),
        jax.ShapeDtypeStruct((N, ND * 8), jnp.float32),
        jax.ShapeDtypeStruct((ND, HID), jnp.float32),
    ],
)


def _s1_body(src_hbm, dst_hbm, pw_hbm, srcrow_hbm, ertab_hbm,
             aout_hbm, bout_hbm, exout_hbm,
             srcv, dstv, pwv, srow, erow, bbuf, exb4, zbuf48, zbuf16,
             acc_a, acc_b,
             semi0, semi1, semi2, semg0, semg1, sems0, sems1, sems2):
    cid = lax.axis_index("c")
    sid = lax.axis_index("s")
    w = sid * 2 + cid
    zv = jnp.zeros((16,), jnp.float32)

    def zinit(rr, _):
        zbuf48[rr, 0:16] = zv
        zbuf48[rr, 16:32] = zv
        zbuf48[rr, 32:48] = zv
        zbuf16[rr, 0:16] = zv
        return 0
    lax.fori_loop(0, ZR, zinit, 0)

    onehot4 = jnp.where(lax.iota(jnp.int32, 16) == 4, 1.0, 0.0).astype(jnp.float32)
    def binit(rr, _):
        bbuf[rr, 0:16] = onehot4
        return 0
    lax.fori_loop(0, CH, binit, 0)

    for g in range(ND):
        def zcp(t, _):
            cidx = sid * ZPT + t
            @pl.when(cidx < NZCH)
            def _():
                pltpu.sync_copy(zbuf48, acc_a.at[pl.ds(cidx * ZR, ZR)])
                pltpu.sync_copy(zbuf16, acc_b.at[pl.ds(cidx * ZR, ZR)])
            return 0
        lax.fori_loop(0, ZPT, zcp, 0)

        @pl.when(sid == 0)
        def _():
            pltpu.sync_copy(zbuf48.at[pl.ds(0, ZREM)],
                            acc_a.at[pl.ds(NZCH * ZR, ZREM)])
            pltpu.sync_copy(zbuf16.at[pl.ds(0, ZREM)],
                            acc_b.at[pl.ds(NZCH * ZR, ZREM)])

        plsc.subcore_barrier()

        ebase = g * E + w * EPT

        def jloop(j, _):
            base = pl.multiple_of(ebase + j * CH, 8)
            di = (
                pltpu.async_copy(src_hbm.at[pl.ds(base, CH)], srcv, semi0),
                pltpu.async_copy(dst_hbm.at[pl.ds(base, CH)], dstv, semi1),
                pltpu.async_copy(pw_hbm.at[pl.ds(base, CH)], pwv, semi2),
            )
            for d in di:
                d.wait()
            dg = (
                pltpu.async_copy(srcrow_hbm.at[srcv], srow, semg0),
                pltpu.async_copy(ertab_hbm.at[dstv], erow, semg1),
            )
            for d in dg:
                d.wait()
            for jj in range(CH // 16):
                rows = lax.iota(jnp.int32, 16) + (jj * 16)
                pwvec = pwv[pl.ds(jj * 16, 16)]
                for h in range(HEADS):
                    ch = jnp.full((16,), h, jnp.int32)
                    elh = plsc.load_gather(srow, [rows, ch + 36])
                    erh = plsc.load_gather(erow, [rows, ch])
                    e = elh + erh
                    e = jnp.maximum(e, 0.2 * e) * pwvec
                    exv = jnp.exp(e)
                    plsc.store_scatter(bbuf, [rows, ch], exv)
                    plsc.store_scatter(exb4, [rows, ch], exv)
            for jj in range(CH // 16):
                sl = pl.ds(jj * 16, 16)
                dstv[sl] = dstv[sl] - (g * N)
            ds = (
                pltpu.async_copy(srow, acc_a.at[dstv], sems0, add=True),
                pltpu.async_copy(bbuf, acc_b.at[dstv], sems1, add=True),
                pltpu.async_copy(exb4, exout_hbm.at[pl.ds(base, CH)], sems2),
            )
            for d in ds:
                d.wait()
            return 0

        lax.fori_loop(0, NCH, jloop, 0)

        plsc.subcore_barrier()

        @pl.when(sid == 0)
        def _():
            pltpu.sync_copy(acc_a, aout_hbm.at[cid, g])
            pltpu.sync_copy(acc_b, bout_hbm.at[cid, g])

        plsc.subcore_barrier()


_s1_call = pl.kernel(
    _s1_body,
    out_type=[
        jax.ShapeDtypeStruct((2, ND, N, 48), jnp.float32),
        jax.ShapeDtypeStruct((2, ND, N, 16), jnp.float32),
        jax.ShapeDtypeStruct((ND * E, HEADS), jnp.float32),
    ],
    mesh=plsc.VectorSubcoreMesh(core_axis_name="c", subcore_axis_name="s"),
    compiler_params=pltpu.CompilerParams(needs_layout_passes=False,
                                         use_tc_tiling_on_sc=False),
    scratch_types=(
        [pltpu.VMEM((CH,), jnp.int32),
         pltpu.VMEM((CH,), jnp.int32),
         pltpu.VMEM((CH,), jnp.float32),
         pltpu.VMEM((CH, 48), jnp.float32),
         pltpu.VMEM((CH, 16), jnp.float32),
         pltpu.VMEM((CH, 16), jnp.float32),
         pltpu.VMEM((CH, HEADS), jnp.float32),
         pltpu.VMEM((ZR, 48), jnp.float32),
         pltpu.VMEM((ZR, 16), jnp.float32),
         pltpu.VMEM_SHARED((N, 48), jnp.float32),
         pltpu.VMEM_SHARED((N, 16), jnp.float32)]
        + [pltpu.SemaphoreType.DMA] * 8
    ),
)


_GB = 1000


def _gate_body(apart_ref, bpart_ref, m_ref, xg1_ref, gw2_ref, gb_ref, c_ref):
    for i in range(ND):
        a = apart_ref[0, i] + apart_ref[1, i]
        b = bpart_ref[0, i] + bpart_ref[1, i]
        sq = a[:, 0:HID]
        xgs = a[:, HID:HID + HEADS]
        exsum = b[:, 0:HEADS]
        deg = b[:, 4:5]
        mx = m_ref[i][None, :] + jnp.log(jnp.maximum(sq, 1e-43)) * (1.0 / TLSE)
        mx = jnp.where(deg > 0.0, mx, 0.0)
        mean4 = xgs / jnp.maximum(deg, 1.0)
        logits = (xg1_ref[i]
                  + jnp.dot(mx, gw2_ref[i], preferred_element_type=jnp.float32)
                  + mean4 + gb_ref[i][None, :])
        g = jax.nn.sigmoid(logits)
        c_ref[i] = g / (4.0 * (exsum + 1e-9))


_gate_call = pl.pallas_call(
    _gate_body,
    grid=(N // _GB,),
    in_specs=[
        pl.BlockSpec((2, ND, _GB, 48), lambda i: (0, 0, i, 0)),
        pl.BlockSpec((2, ND, _GB, 16), lambda i: (0, 0, i, 0)),
        pl.BlockSpec((ND, HID), lambda i: (0, 0)),
        pl.BlockSpec((ND, _GB, HEADS), lambda i: (0, i, 0)),
        pl.BlockSpec((ND, HID, HEADS), lambda i: (0, 0, 0)),
        pl.BlockSpec((ND, HEADS), lambda i: (0, 0)),
    ],
    out_specs=[pl.BlockSpec((ND, _GB, HEADS), lambda i: (0, i, 0))],
    out_shape=[jax.ShapeDtypeStruct((ND, N, HEADS), jnp.float32)],
)


def _s2_body(src_hbm, dst_hbm, ztab_hbm, ctab_hbm, exbuf_hbm,
             accout_hbm,
             srcv, dstv, zrow, crow, exrow, zbuf32, acc,
             semi0, semi1, semi2, semg0, semg1, sems0):
    cid = lax.axis_index("c")
    sid = lax.axis_index("s")
    w = sid * 2 + cid
    zv = jnp.zeros((16,), jnp.float32)

    def zinit(rr, _):
        zbuf32[rr, 0:16] = zv
        zbuf32[rr, 16:32] = zv
        return 0
    lax.fori_loop(0, ZR, zinit, 0)

    for g in range(ND):
        def zcp(t, _):
            cidx = sid * ZPT + t
            @pl.when(cidx < NZCH)
            def _():
                pltpu.sync_copy(zbuf32, acc.at[pl.ds(cidx * ZR, ZR)])
            return 0
        lax.fori_loop(0, ZPT, zcp, 0)

        @pl.when(sid == 0)
        def _():
            pltpu.sync_copy(zbuf32.at[pl.ds(0, ZREM)],
                            acc.at[pl.ds(NZCH * ZR, ZREM)])

        plsc.subcore_barrier()

        ebase = g * E + w * EPT

        def jloop(j, _):
            base = pl.multiple_of(ebase + j * CH, 8)
            di = (
                pltpu.async_copy(src_hbm.at[pl.ds(base, CH)], srcv, semi0),
                pltpu.async_copy(dst_hbm.at[pl.ds(base, CH)], dstv, semi1),
                pltpu.async_copy(exbuf_hbm.at[pl.ds(base, CH)], exrow, semi2),
            )
            for d in di:
                d.wait()
            dg = (
                pltpu.async_copy(ztab_hbm.at[srcv], zrow, semg0),
                pltpu.async_copy(ctab_hbm.at[dstv], crow, semg1),
            )
            for d in dg:
                d.wait()
            for jj in range(CH // 16):
                rows = lax.iota(jnp.int32, 16) + (jj * 16)
                csum = jnp.zeros((16,), jnp.float32)
                for h in range(HEADS):
                    ch = jnp.full((16,), h, jnp.int32)
                    csum = csum + (plsc.load_gather(exrow, [rows, ch])
                                   * plsc.load_gather(crow, [rows, ch]))
                for col in range(HID):
                    cc = jnp.full((16,), col, jnp.int32)
                    v = plsc.load_gather(zrow, [rows, cc])
                    plsc.store_scatter(zrow, [rows, cc], v * csum)
            for jj in range(CH // 16):
                sl = pl.ds(jj * 16, 16)
                dstv[sl] = dstv[sl] - (g * N)
            pltpu.async_copy(zrow, acc.at[dstv], sems0, add=True).wait()
            return 0

        lax.fori_loop(0, NCH, jloop, 0)

        plsc.subcore_barrier()

        @pl.when(sid == 0)
        def _():
            pltpu.sync_copy(acc, accout_hbm.at[cid, g])

        plsc.subcore_barrier()


_s2_call = pl.kernel(
    _s2_body,
    out_type=[jax.ShapeDtypeStruct((2, ND, N, HID), jnp.float32)],
    mesh=plsc.VectorSubcoreMesh(core_axis_name="c", subcore_axis_name="s"),
    compiler_params=pltpu.CompilerParams(needs_layout_passes=False,
                                         use_tc_tiling_on_sc=False),
    scratch_types=(
        [pltpu.VMEM((CH,), jnp.int32),
         pltpu.VMEM((CH,), jnp.int32),
         pltpu.VMEM((CH, HID), jnp.float32),
         pltpu.VMEM((CH, 16), jnp.float32),
         pltpu.VMEM((CH, HEADS), jnp.float32),
         pltpu.VMEM((ZR, HID), jnp.float32),
         pltpu.VMEM_SHARED((N, HID), jnp.float32)]
        + [pltpu.SemaphoreType.DMA] * 6
    ),
)


def _stats_body(proj_ref, accp_ref, h_ref, stats_ref):
    i = pl.program_id(0)
    parts = [proj_ref[...]]
    for g in range(ND):
        parts.append(accp_ref[0, g] + accp_ref[1, g])
    h = jnp.concatenate(parts, axis=1)
    h_ref[0] = h

    @pl.when(i == 0)
    def _():
        stats_ref[...] = jnp.zeros((2, C_OUT), jnp.float32)
    stats_ref[0:1] = stats_ref[0:1] + jnp.sum(h, axis=0, keepdims=True)
    stats_ref[1:2] = stats_ref[1:2] + jnp.sum(h * h, axis=0, keepdims=True)


_stats_call = pl.pallas_call(
    _stats_body,
    grid=(N // _GB,),
    in_specs=[
        pl.BlockSpec((_GB, HID), lambda i: (i, 0)),
        pl.BlockSpec((2, ND, _GB, HID), lambda i: (0, 0, i, 0)),
    ],
    out_specs=[
        pl.BlockSpec((1, _GB, C_OUT), lambda i: (0, i, 0)),
        pl.BlockSpec((2, C_OUT), lambda i: (0, 0)),
    ],
    out_shape=[
        jax.ShapeDtypeStruct((1, N, C_OUT), jnp.float32),
        jax.ShapeDtypeStruct((2, C_OUT), jnp.float32),
    ],
)


def _bn_body(h_ref, stats_ref, gamma_ref, beta_ref, out_ref):
    mean = stats_ref[0:1] * (1.0 / N)
    var = stats_ref[1:2] * (1.0 / N) - mean * mean
    hn = (h_ref[0] - mean) * lax.rsqrt(var + 1e-5)
    out_ref[0] = hn * gamma_ref[...][None, :] + beta_ref[...][None, :]


_bn_call = pl.pallas_call(
    _bn_body,
    grid=(N // _GB,),
    in_specs=[
        pl.BlockSpec((1, _GB, C_OUT), lambda i: (0, i, 0)),
        pl.BlockSpec((2, C_OUT), lambda i: (0, 0)),
        pl.BlockSpec((C_OUT,), lambda i: (0,)),
        pl.BlockSpec((C_OUT,), lambda i: (0,)),
    ],
    out_specs=[pl.BlockSpec((1, _GB, C_OUT), lambda i: (0, i, 0))],
    out_shape=[jax.ShapeDtypeStruct((1, N, C_OUT), jnp.float32)],
)


def kernel(x, pre_w0, pre_w1, pre_w2, W_proj, a_l, a_r, Wm, gate_w, gate_b,
           bn_gamma, bn_beta, edge_index0, edge_index1, edge_index2):
    alt = jnp.transpose(a_l, (0, 2, 1))
    art = jnp.transpose(a_r, (0, 2, 1))
    gw1 = gate_w[:, 0:FEAT, :]
    gw2 = gate_w[:, FEAT:FEAT + HID, :]
    gw3 = gate_w[:, FEAT + HID:, :]

    wcat = jnp.concatenate(
        [W_proj, Wm[0], Wm[1], Wm[2],
         gw3[0], gw1[0], gw3[1], gw1[1], gw3[2], gw1[2]], axis=1)
    w2cat = jnp.concatenate(
        [alt[0], art[0], alt[1], art[1], alt[2], art[2]], axis=1)

    y, qcat, ea, m = _pre_call(x, wcat, w2cat)
    proj = y[:, 0:HID]

    zpad8 = jnp.zeros((N, 8), jnp.float32)
    srcrow = jnp.concatenate([
        jnp.concatenate([qcat[:, HID * g:HID * (g + 1)],
                         y[:, FEAT + 8 * g:FEAT + 8 * g + 4],
                         ea[:, 8 * g:8 * g + 4], zpad8], axis=1)
        for g in range(ND)], axis=0)
    zpad12 = jnp.zeros((N, 12), jnp.float32)
    ertab = jnp.concatenate([
        jnp.concatenate([ea[:, 8 * g + 4:8 * g + 8], zpad12], axis=1)
        for g in range(ND)], axis=0)
    xg1 = jnp.stack([y[:, FEAT + 8 * g + 4:FEAT + 8 * g + 8] for g in range(ND)])

    offs = jnp.arange(ND, dtype=jnp.int32)[:, None] * N
    src_all = (jnp.stack([edge_index0[0], edge_index1[0], edge_index2[0]]) + offs).reshape(-1)
    dst_all = (jnp.stack([edge_index0[1], edge_index1[1], edge_index2[1]]) + offs).reshape(-1)
    pw_all = jnp.concatenate([pre_w0, pre_w1, pre_w2])

    apart, bpart, exbuf = _s1_call(src_all, dst_all, pw_all, srcrow, ertab)

    (coeff,) = _gate_call(apart, bpart, m, xg1, gw2, gate_b)
    ctab = jnp.concatenate(
        [coeff.reshape(ND * N, HEADS), jnp.zeros((ND * N, 12), jnp.float32)], axis=1)
    ztab = jnp.tile(proj, (ND, 1))

    (accp,) = _s2_call(src_all, dst_all, ztab, ctab, exbuf)

    h, stats = _stats_call(proj, accp)
    (out,) = _bn_call(h, stats, bn_gamma, bn_beta)
    return out
